# baseline (device time: 19696 ns/iter reference)
import jax
import jax.numpy as jnp
from jax import lax
from jax.experimental import pallas as pl
from jax.experimental.pallas import tpu as pltpu

N_DEV = 8
M = 256
NCOLS = 128
GCOLS = NCOLS // N_DEV
T = M // 8


def _roll(x, shift, axis):
    return pltpu.roll(x, shift, axis)


def _bitswap(y):
    row = lax.broadcasted_iota(jnp.int32, (M, 1), 0)
    lane = lax.broadcasted_iota(jnp.int32, (1, NCOLS), 1)
    for b in range(3):
        sb = T << b
        lb = GCOLS << b
        rb = (row >> (5 + b)) & 1
        cb = (lane >> (4 + b)) & 1
        same = rb == cb
        r1 = _roll(_roll(y, NCOLS - lb, 1), sb, 0)
        r2 = _roll(_roll(y, lb, 1), M - sb, 0)
        y = jnp.where(same, y, jnp.where(rb == 1, r1, r2))
    return y


def _zstage(y, k, j, rank_map):
    up_full = (rank_map & k) == 0
    if j <= 16 or j >= M:
        jr = j // 8 if j >= M else j
        g = M // (2 * jr)
        yr = y.reshape(g, 2 * jr, NCOLS)
        lo, hi = yr[:, :jr, :], yr[:, jr:, :]
        mn, mx = jnp.minimum(lo, hi), jnp.maximum(lo, hi)
        up = jnp.broadcast_to(up_full, (M, NCOLS)).reshape(g, 2 * jr, NCOLS)[:, :jr, :]
        return jnp.concatenate(
            [jnp.where(up, mn, mx), jnp.where(up, mx, mn)], axis=1
        ).reshape(M, NCOLS)
    shift = j // 2
    lane = lax.broadcasted_iota(jnp.int32, (1, NCOLS), 1)
    is_low = (lane & shift) == 0
    partner = jnp.where(
        is_low, _roll(y, NCOLS - shift, 1), _roll(y, shift, 1)
    )
    keep_min = up_full == is_low
    return jnp.where(keep_min, jnp.minimum(y, partner), jnp.maximum(y, partner))


def _rank_maps(row0):
    row = lax.broadcasted_iota(jnp.int32, (M, 1), 0)
    lane = lax.broadcasted_iota(jnp.int32, (1, NCOLS), 1)
    rank_low = (row & (T - 1)) + ((lane >> 4) & 7) * T
    return row0 + rank_low, (row >> 5) * M + rank_low


def _sort_local(y, rank1):
    k = 2
    while k <= M:
        j = k // 2
        while j >= 1:
            y = _zstage(y, k, j, rank1)
            j //= 2
        k *= 2
    return y


def _merge(y, rank2):
    for k in (2 * M, 4 * M, 8 * M):
        j = k // 2
        while j >= 1:
            y = _zstage(y, k, j, rank2)
            j //= 2
    return y


def kernel(x):
    assert x.shape == (M, NCOLS)

    def body(
        x_ref, out_ref,
        stage1, recv1, stage2, recv2,
        send_sems1, recv_sems1, send_sems2, recv_sems2,
    ):
        my = lax.axis_index("i")
        rank1, rank2 = _rank_maps(my * M)

        barrier_sem = pltpu.get_barrier_semaphore()
        for off in range(1, N_DEV):
            pl.semaphore_signal(
                barrier_sem, inc=1,
                device_id=(my ^ off,), device_id_type=pl.DeviceIdType.MESH,
            )
        pl.semaphore_wait(barrier_sem, N_DEV - 1)

        z = _sort_local(_bitswap(x_ref[...]), rank1)
        stage1[...] = z.reshape(N_DEV, T, NCOLS)

        recv1[my] = stage1[my]
        rdmas = []
        for off in range(1, N_DEV):
            tgt = my ^ off
            rdma = pltpu.make_async_remote_copy(
                src_ref=stage1.at[tgt],
                dst_ref=recv1.at[my],
                send_sem=send_sems1.at[off - 1],
                recv_sem=recv_sems1.at[off - 1],
                device_id=(tgt,),
                device_id_type=pl.DeviceIdType.MESH,
            )
            rdma.start()
            rdmas.append(rdma)
        for rdma in rdmas[-(N_DEV - 1):]:
            rdma.wait_recv()

        w = _merge(recv1[...].reshape(M, NCOLS), rank2)
        stage2[...] = w.reshape(N_DEV, T, NCOLS)

        recv2[my] = stage2[my]
        for off in range(1, N_DEV):
            tgt = my ^ off
            rdma = pltpu.make_async_remote_copy(
                src_ref=stage2.at[tgt],
                dst_ref=recv2.at[my],
                send_sem=send_sems2.at[off - 1],
                recv_sem=recv_sems2.at[off - 1],
                device_id=(tgt,),
                device_id_type=pl.DeviceIdType.MESH,
            )
            rdma.start()
            rdmas.append(rdma)
        for rdma in rdmas[-(N_DEV - 1):]:
            rdma.wait_recv()

        out_ref[...] = _bitswap(recv2[...].reshape(M, NCOLS))
        for rdma in rdmas:
            rdma.wait_send()

    return pl.pallas_call(
        body,
        out_shape=jax.ShapeDtypeStruct((M, NCOLS), x.dtype),
        in_specs=[pl.BlockSpec(memory_space=pltpu.VMEM)],
        out_specs=pl.BlockSpec(memory_space=pltpu.VMEM),
        scratch_shapes=[
            pltpu.VMEM((N_DEV, T, NCOLS), x.dtype),
            pltpu.VMEM((N_DEV, T, NCOLS), x.dtype),
            pltpu.VMEM((N_DEV, T, NCOLS), x.dtype),
            pltpu.VMEM((N_DEV, T, NCOLS), x.dtype),
            pltpu.SemaphoreType.DMA((N_DEV - 1,)),
            pltpu.SemaphoreType.DMA((N_DEV - 1,)),
            pltpu.SemaphoreType.DMA((N_DEV - 1,)),
            pltpu.SemaphoreType.DMA((N_DEV - 1,)),
        ],
        compiler_params=pltpu.CompilerParams(collective_id=0),
    )(x)


# device time: 19405 ns/iter; 1.0150x vs baseline; 1.0150x over previous
import jax
import jax.numpy as jnp
from jax import lax
from jax.experimental import pallas as pl
from jax.experimental.pallas import tpu as pltpu

N_DEV = 8
M = 256
NCOLS = 128
GCOLS = NCOLS // N_DEV
T = M // 8


def _roll(x, shift):
    return pltpu.roll(x, shift, len(x.shape) - 1)


def _sterm(s_term, ndim):
    if isinstance(s_term, int) or getattr(s_term, "ndim", 0) == 0:
        return s_term
    return s_term.reshape((N_DEV,) + (1,) * (ndim - 1))


def _packed_stage(y, k, j, s_term):
    if j >= M:
        gd = j // M
        ng = N_DEV // (2 * gd)
        yr = y.reshape(ng, 2 * gd, T, NCOLS)
        lo, hi = yr[:, :gd], yr[:, gd:]
        mn, mx = jnp.minimum(lo, hi), jnp.maximum(lo, hi)
        q = lax.broadcasted_iota(jnp.int32, (ng, 1, 1, 1), 0)
        u = lax.broadcasted_iota(jnp.int32, (1, gd, 1, 1), 1)
        up = (((q * 2 * gd + u) * M) & k) == 0
        return jnp.concatenate(
            [jnp.where(up, mn, mx), jnp.where(up, mx, mn)], axis=1
        ).reshape(N_DEV, T, NCOLS)
    if j >= 8:
        jt = j // 8
        gt = T // (2 * jt)
        yr = y.reshape(N_DEV, gt, 2 * jt, NCOLS)
        lo, hi = yr[:, :, :jt], yr[:, :, jt:]
        mn, mx = jnp.minimum(lo, hi), jnp.maximum(lo, hi)
        gidx = lax.broadcasted_iota(jnp.int32, (1, gt, 1, 1), 1)
        st = _sterm(s_term, 4)
        up = ((st + gidx * 2 * jt * 8) & k) == 0
        return jnp.concatenate(
            [jnp.where(up, mn, mx), jnp.where(up, mx, mn)], axis=2
        ).reshape(N_DEV, T, NCOLS)
    shift = GCOLS * j
    lane = lax.broadcasted_iota(jnp.int32, (1, 1, NCOLS), 2)
    u = lane // GCOLS
    t = lax.broadcasted_iota(jnp.int32, (1, T, 1), 1)
    st = _sterm(s_term, 3)
    is_low = (u & j) == 0
    partner = jnp.where(is_low, _roll(y, NCOLS - shift), _roll(y, shift))
    up = ((st + t * 8 + u) & k) == 0
    keep_min = up == is_low
    return jnp.where(keep_min, jnp.minimum(y, partner), jnp.maximum(y, partner))


def _packed_sort_local(y, row0):
    k = 2
    while k <= M:
        j = k // 2
        while j >= 1:
            y = _packed_stage(y, k, j, 0 if k < M else row0)
            j //= 2
        k *= 2
    return y


def _packed_merge(y):
    s = lax.broadcasted_iota(jnp.int32, (N_DEV,), 0) * M
    for k in (2 * M, 4 * M, 8 * M):
        j = k // 2
        while j >= 1:
            y = _packed_stage(y, k, j, s)
            j //= 2
    return y


def _pack(xv):
    xr = xv.reshape(T, 8, NCOLS)
    slabs = []
    for d in range(N_DEV):
        rows = [xr[:, u, d * GCOLS:(d + 1) * GCOLS] for u in range(8)]
        slabs.append(jnp.concatenate(rows, axis=1)[None])
    return jnp.concatenate(slabs, axis=0)


def _unpack(slabs):
    planes = []
    for u in range(8):
        row_u = jnp.concatenate(
            [s[:, u * GCOLS:(u + 1) * GCOLS] for s in slabs], axis=1
        )
        planes.append(row_u[:, None, :])
    return jnp.concatenate(planes, axis=1).reshape(M, NCOLS)


def kernel(x):
    assert x.shape == (M, NCOLS)

    def body(
        x_ref, out_ref,
        stage1, recv1, stage2, recv2,
        send_sems1, recv_sems1, send_sems2, recv_sems2,
    ):
        my = lax.axis_index("i")

        barrier_sem = pltpu.get_barrier_semaphore()
        for off in range(1, N_DEV):
            pl.semaphore_signal(
                barrier_sem, inc=1,
                device_id=(my ^ off,), device_id_type=pl.DeviceIdType.MESH,
            )
        pl.semaphore_wait(barrier_sem, N_DEV - 1)

        y = _packed_sort_local(_pack(x_ref[...]), my * M)
        stage1[...] = y

        recv1[my] = stage1[my]
        rdmas = []
        for off in range(1, N_DEV):
            tgt = my ^ off
            rdma = pltpu.make_async_remote_copy(
                src_ref=stage1.at[tgt],
                dst_ref=recv1.at[my],
                send_sem=send_sems1.at[off - 1],
                recv_sem=recv_sems1.at[off - 1],
                device_id=(tgt,),
                device_id_type=pl.DeviceIdType.MESH,
            )
            rdma.start()
            rdmas.append(rdma)
        for rdma in rdmas[-(N_DEV - 1):]:
            rdma.wait_recv()

        stage2[...] = _packed_merge(recv1[...])

        recv2[my] = stage2[my]
        for off in range(1, N_DEV):
            tgt = my ^ off
            rdma = pltpu.make_async_remote_copy(
                src_ref=stage2.at[tgt],
                dst_ref=recv2.at[my],
                send_sem=send_sems2.at[off - 1],
                recv_sem=recv_sems2.at[off - 1],
                device_id=(tgt,),
                device_id_type=pl.DeviceIdType.MESH,
            )
            rdma.start()
            rdmas.append(rdma)
        for rdma in rdmas[-(N_DEV - 1):]:
            rdma.wait_recv()

        out_ref[...] = _unpack([recv2[d] for d in range(N_DEV)])
        for rdma in rdmas:
            rdma.wait_send()

    return pl.pallas_call(
        body,
        out_shape=jax.ShapeDtypeStruct((M, NCOLS), x.dtype),
        in_specs=[pl.BlockSpec(memory_space=pltpu.VMEM)],
        out_specs=pl.BlockSpec(memory_space=pltpu.VMEM),
        scratch_shapes=[
            pltpu.VMEM((N_DEV, T, NCOLS), x.dtype),
            pltpu.VMEM((N_DEV, T, NCOLS), x.dtype),
            pltpu.VMEM((N_DEV, T, NCOLS), x.dtype),
            pltpu.VMEM((N_DEV, T, NCOLS), x.dtype),
            pltpu.SemaphoreType.DMA((N_DEV - 1,)),
            pltpu.SemaphoreType.DMA((N_DEV - 1,)),
            pltpu.SemaphoreType.DMA((N_DEV - 1,)),
            pltpu.SemaphoreType.DMA((N_DEV - 1,)),
        ],
        compiler_params=pltpu.CompilerParams(collective_id=0),
    )(x)
